# baseline (speedup 1.0000x reference)
import jax
import jax.numpy as jnp
from jax.experimental import pallas as pl
from jax.experimental.pallas import tpu as pltpu

_BB = 2
_NBUF = 8
_W = 4


def _body(patch_hbm, table_hbm, out_hbm, buf, table_v, in_sem, out_sem, tab_sem):
    i = pl.program_id(0)
    n = pl.num_programs(0)
    s = i % _NBUF

    @pl.when(i == 0)
    def _prologue():
        pltpu.make_async_copy(table_hbm, table_v, tab_sem).start()
        for j in range(_W):
            pltpu.make_async_copy(
                patch_hbm.at[pl.ds(j * _BB, _BB)], buf.at[j], in_sem.at[j]
            ).start()
        pltpu.make_async_copy(table_hbm, table_v, tab_sem).wait()

    pltpu.make_async_copy(
        patch_hbm.at[pl.ds(i * _BB, _BB)], buf.at[s], in_sem.at[s]
    ).wait()
    buf[s] = buf[s] + table_v[...]
    pltpu.make_async_copy(
        buf.at[s], out_hbm.at[pl.ds(i * _BB, _BB)], out_sem.at[s]
    ).start()

    j = i + _W
    t = j % _NBUF

    @pl.when(j < n)
    def _lookahead():
        @pl.when(j >= _NBUF)
        def _free_slot():
            pltpu.make_async_copy(
                buf.at[t], out_hbm.at[pl.ds((j - _NBUF) * _BB, _BB)], out_sem.at[t]
            ).wait()

        pltpu.make_async_copy(
            patch_hbm.at[pl.ds(j * _BB, _BB)], buf.at[t], in_sem.at[t]
        ).start()

    @pl.when(i == n - 1)
    def _epilogue():
        for k in range(_NBUF):
            step = n - _NBUF + k
            pltpu.make_async_copy(
                buf.at[step % _NBUF],
                out_hbm.at[pl.ds(step * _BB, _BB)],
                out_sem.at[step % _NBUF],
            ).wait()


def kernel(patch, pos_table):
    B, P, D = patch.shape
    n = B // _BB
    return pl.pallas_call(
        _body,
        grid=(n,),
        in_specs=[
            pl.BlockSpec(memory_space=pl.ANY),
            pl.BlockSpec(memory_space=pl.ANY),
        ],
        out_specs=pl.BlockSpec(memory_space=pl.ANY),
        out_shape=jax.ShapeDtypeStruct((B, P, D), patch.dtype),
        scratch_shapes=[
            pltpu.VMEM((_NBUF, _BB, P, D), patch.dtype),
            pltpu.VMEM((P, D), pos_table.dtype),
            pltpu.SemaphoreType.DMA((_NBUF,)),
            pltpu.SemaphoreType.DMA((_NBUF,)),
            pltpu.SemaphoreType.DMA,
        ],
        compiler_params=pltpu.CompilerParams(
            dimension_semantics=("arbitrary",),
        ),
    )(patch, pos_table)
